# initial kernel scaffold (unmeasured)
import jax
import jax.numpy as jnp
from jax import lax
from jax.experimental import pallas as pl
from jax.experimental.pallas import tpu as pltpu

N_DEV = 4
M_PER = 1024
K = 4096
N_PER = 512


def kernel(x, w_mat):
    def body(x_hbm, w_hbm, out_ref, xs, xb, ws, yb, rb,
             load_sem, send_sems, recv_sems):
        me = lax.axis_index("i")

        x_load = pltpu.make_async_copy(x_hbm, xs, load_sem)
        x_load.start()

        barrier_sem = pltpu.get_barrier_semaphore()
        for d in (1, 2, 3):
            pl.semaphore_signal(
                barrier_sem, inc=1,
                device_id=((me + d) % N_DEV,),
                device_id_type=pl.DeviceIdType.MESH,
            )
        pl.semaphore_wait(barrier_sem, 3)

        x_load.wait()
        for c in range(4):
            xb[c * 256:(c + 1) * 256, :] = (
                xs[c * 256:(c + 1) * 256, :].astype(jnp.bfloat16))

        sends = []
        for d in (1, 2, 3, 4):
            peer = (me + d) % N_DEV
            w_load = pltpu.make_async_copy(
                w_hbm.at[:, pl.ds(peer * N_PER, N_PER)], ws, load_sem)
            w_load.start()
            w_load.wait()
            yj = jnp.dot(xb[:, :], ws[:, :].astype(jnp.bfloat16),
                         preferred_element_type=jnp.float32)
            yj = yj * jax.nn.sigmoid(yj)
            if d < 4:
                yb[d - 1] = yj.astype(jnp.bfloat16)
                rdma = pltpu.make_async_remote_copy(
                    src_ref=yb.at[d - 1],
                    dst_ref=rb.at[3 - d],
                    send_sem=send_sems.at[d - 1],
                    recv_sem=recv_sems.at[3 - d],
                    device_id=(peer,),
                    device_id_type=pl.DeviceIdType.MESH,
                )
                rdma.start()
                sends.append(rdma)
            else:
                out_ref[pl.ds(me * M_PER, M_PER), :] = yj

        for s in (3, 2, 1):
            recv = pltpu.make_async_remote_copy(
                src_ref=yb.at[0],
                dst_ref=rb.at[s - 1],
                send_sem=send_sems.at[0],
                recv_sem=recv_sems.at[s - 1],
                device_id=(me,),
                device_id_type=pl.DeviceIdType.MESH,
            )
            recv.wait_recv()
            origin = (me + s) % N_DEV
            out_ref[pl.ds(origin * M_PER, M_PER), :] = (
                rb[s - 1].astype(jnp.float32))

        for rdma in sends:
            rdma.wait_send()

    out_shape = jax.ShapeDtypeStruct((N_DEV * M_PER, N_PER), jnp.float32)
    return pl.pallas_call(
        body,
        out_shape=out_shape,
        in_specs=[
            pl.BlockSpec(memory_space=pltpu.ANY),
            pl.BlockSpec(memory_space=pltpu.ANY),
        ],
        out_specs=pl.BlockSpec(memory_space=pltpu.VMEM),
        scratch_shapes=[
            pltpu.VMEM((M_PER, K), jnp.float32),
            pltpu.VMEM((M_PER, K), jnp.bfloat16),
            pltpu.VMEM((K, N_PER), jnp.float32),
            pltpu.VMEM((3, M_PER, N_PER), jnp.bfloat16),
            pltpu.VMEM((3, M_PER, N_PER), jnp.bfloat16),
            pltpu.SemaphoreType.DMA,
            pltpu.SemaphoreType.DMA((3,)),
            pltpu.SemaphoreType.DMA((3,)),
        ],
        compiler_params=pltpu.CompilerParams(collective_id=0),
    )(x, w_mat)


# baseline (device time: 60837 ns/iter reference)
import jax
import jax.numpy as jnp
from jax import lax
from jax.experimental import pallas as pl
from jax.experimental.pallas import tpu as pltpu

N_DEV = 4
M_PER = 1024
K = 4096
N_PER = 512


def kernel(x, w_mat):
    def body(x_hbm, w_hbm, out_ref, xs, xb, ws, yb, rb,
             load_sem, send_sems, recv_sems):
        me = lax.axis_index("i")

        x_load = pltpu.make_async_copy(x_hbm, xs, load_sem)
        x_load.start()

        barrier_sem = pltpu.get_barrier_semaphore()
        for d in (1, 2, 3):
            pl.semaphore_signal(
                barrier_sem, inc=1,
                device_id=((me + d) % N_DEV,),
                device_id_type=pl.DeviceIdType.MESH,
            )
        pl.semaphore_wait(barrier_sem, 3)

        x_load.wait()
        for c in range(4):
            xb[c * 256:(c + 1) * 256, :] = (
                xs[c * 256:(c + 1) * 256, :].astype(jnp.bfloat16))

        sends = []
        for d in (1, 2, 3, 4):
            peer = (me + d) % N_DEV
            w_load = pltpu.make_async_copy(
                w_hbm.at[:, pl.ds(peer * N_PER, N_PER)], ws, load_sem)
            w_load.start()
            w_load.wait()
            yj = jnp.dot(xb[:, :], ws[:, :].astype(jnp.bfloat16),
                         preferred_element_type=jnp.float32)
            yj = yj * jax.nn.sigmoid(yj)
            if d < 4:
                yb[d - 1] = yj.astype(jnp.bfloat16)
                rdma = pltpu.make_async_remote_copy(
                    src_ref=yb.at[d - 1],
                    dst_ref=rb.at[3 - d],
                    send_sem=send_sems.at[d - 1],
                    recv_sem=recv_sems.at[3 - d],
                    device_id=(peer,),
                    device_id_type=pl.DeviceIdType.MESH,
                )
                rdma.start()
                sends.append(rdma)
            else:
                out_ref[pl.ds(me * M_PER, M_PER), :] = yj

        for s in (3, 2, 1):
            recv = pltpu.make_async_remote_copy(
                src_ref=yb.at[0],
                dst_ref=rb.at[s - 1],
                send_sem=send_sems.at[0],
                recv_sem=recv_sems.at[s - 1],
                device_id=(me,),
                device_id_type=pl.DeviceIdType.MESH,
            )
            recv.wait_recv()
            origin = (me + s) % N_DEV
            out_ref[pl.ds(origin * M_PER, M_PER), :] = (
                rb[s - 1].astype(jnp.float32))

        for rdma in sends:
            rdma.wait_send()

    out_shape = jax.ShapeDtypeStruct((N_DEV * M_PER, N_PER), jnp.float32)
    return pl.pallas_call(
        body,
        out_shape=out_shape,
        in_specs=[
            pl.BlockSpec(memory_space=pltpu.MemorySpace.HBM),
            pl.BlockSpec(memory_space=pltpu.MemorySpace.HBM),
        ],
        out_specs=pl.BlockSpec(memory_space=pltpu.VMEM),
        scratch_shapes=[
            pltpu.VMEM((M_PER, K), jnp.float32),
            pltpu.VMEM((M_PER, K), jnp.bfloat16),
            pltpu.VMEM((K, N_PER), jnp.float32),
            pltpu.VMEM((3, M_PER, N_PER), jnp.bfloat16),
            pltpu.VMEM((3, M_PER, N_PER), jnp.bfloat16),
            pltpu.SemaphoreType.DMA,
            pltpu.SemaphoreType.DMA((3,)),
            pltpu.SemaphoreType.DMA((3,)),
        ],
        compiler_params=pltpu.CompilerParams(
            collective_id=0,
            vmem_limit_bytes=60 * 1024 * 1024,
        ),
    )(x, w_mat)


# device time: 56455 ns/iter; 1.0776x vs baseline; 1.0776x over previous
import jax
import jax.numpy as jnp
from jax import lax
from jax.experimental import pallas as pl
from jax.experimental.pallas import tpu as pltpu

N_DEV = 4
M_PER = 1024
K = 4096
N_PER = 512
XC = M_PER // 4


def kernel(x, w_mat):
    def body(x_hbm, w_hbm, out_ref, xs, xb, ws, yb, rb,
             xsems, wsems, send_sems, recv_sems):
        me = lax.axis_index("i")

        def x_copy(c):
            return pltpu.make_async_copy(
                x_hbm.at[pl.ds(c * XC, XC)], xs.at[c % 2], xsems.at[c % 2])

        def w_copy(d):
            peer = (me + d) % N_DEV
            buf = (d - 1) % 2
            return pltpu.make_async_copy(
                w_hbm.at[:, pl.ds(peer * N_PER, N_PER)],
                ws.at[buf], wsems.at[buf])

        x_copy(0).start()
        w_copy(1).start()

        barrier_sem = pltpu.get_barrier_semaphore()
        for d in (1, 2, 3):
            pl.semaphore_signal(
                barrier_sem, inc=1,
                device_id=((me + d) % N_DEV,),
                device_id_type=pl.DeviceIdType.MESH,
            )
        pl.semaphore_wait(barrier_sem, 3)

        for c in range(4):
            if c + 1 < 4:
                x_copy(c + 1).start()
            x_copy(c).wait()
            xb[c * XC:(c + 1) * XC, :] = xs[c % 2].astype(jnp.bfloat16)

        sends = []
        for d in (1, 2, 3, 4):
            if d < 4:
                w_copy(d + 1).start()
            w_copy(d).wait()
            buf = (d - 1) % 2
            yj = jnp.dot(xb[:, :], ws[buf].astype(jnp.bfloat16),
                         preferred_element_type=jnp.float32)
            yj = yj * jax.nn.sigmoid(yj)
            if d < 4:
                yb[d - 1] = yj.astype(jnp.bfloat16)
                rdma = pltpu.make_async_remote_copy(
                    src_ref=yb.at[d - 1],
                    dst_ref=rb.at[3 - d],
                    send_sem=send_sems.at[d - 1],
                    recv_sem=recv_sems.at[3 - d],
                    device_id=((me + d) % N_DEV,),
                    device_id_type=pl.DeviceIdType.MESH,
                )
                rdma.start()
                sends.append(rdma)
            else:
                out_ref[pl.ds(me * M_PER, M_PER), :] = yj

        for s in (3, 2, 1):
            recv = pltpu.make_async_remote_copy(
                src_ref=yb.at[0],
                dst_ref=rb.at[s - 1],
                send_sem=send_sems.at[0],
                recv_sem=recv_sems.at[s - 1],
                device_id=(me,),
                device_id_type=pl.DeviceIdType.MESH,
            )
            recv.wait_recv()
            origin = (me + s) % N_DEV
            out_ref[pl.ds(origin * M_PER, M_PER), :] = (
                rb[s - 1].astype(jnp.float32))

        for rdma in sends:
            rdma.wait_send()

    out_shape = jax.ShapeDtypeStruct((N_DEV * M_PER, N_PER), jnp.float32)
    return pl.pallas_call(
        body,
        out_shape=out_shape,
        in_specs=[
            pl.BlockSpec(memory_space=pltpu.MemorySpace.HBM),
            pl.BlockSpec(memory_space=pltpu.MemorySpace.HBM),
        ],
        out_specs=pl.BlockSpec(memory_space=pltpu.VMEM),
        scratch_shapes=[
            pltpu.VMEM((2, XC, K), jnp.float32),
            pltpu.VMEM((M_PER, K), jnp.bfloat16),
            pltpu.VMEM((2, K, N_PER), jnp.float32),
            pltpu.VMEM((3, M_PER, N_PER), jnp.bfloat16),
            pltpu.VMEM((3, M_PER, N_PER), jnp.bfloat16),
            pltpu.SemaphoreType.DMA((2,)),
            pltpu.SemaphoreType.DMA((2,)),
            pltpu.SemaphoreType.DMA((3,)),
            pltpu.SemaphoreType.DMA((3,)),
        ],
        compiler_params=pltpu.CompilerParams(
            collective_id=0,
            vmem_limit_bytes=60 * 1024 * 1024,
        ),
    )(x, w_mat)
